# Initial kernel scaffold; baseline (speedup 1.0000x reference)
import jax
import jax.numpy as jnp
from jax.experimental import pallas as pl


def kernel(x, weights, bias, conns):
    raise NotImplementedError("write your pallas kernel here")


# speedup vs baseline: 4.7709x; 4.7709x over previous
import functools

import jax
import jax.numpy as jnp
from jax import lax
from jax.experimental import pallas as pl
from jax.experimental.pallas import tpu as pltpu
from jax.experimental.pallas import tpu_sc as plsc

INPUTSIZE = 2048
OUTPUTSIZE = 1024
BATCH = 2048
NCONN = 16
KTOT = 32

NW = 32
O_PER_W = OUTPUTSIZE // NW
WCHUNK = O_PER_W * INPUTSIZE


def _sc_build_wt(conns_prep, weights_prep):
    mesh = plsc.VectorSubcoreMesh(core_axis_name="c", subcore_axis_name="s")

    @functools.partial(
        pl.kernel,
        mesh=mesh,
        out_type=jax.ShapeDtypeStruct((OUTPUTSIZE * INPUTSIZE,), jnp.float32),
        scratch_types=[
            pltpu.VMEM((KTOT, O_PER_W), jnp.int32),
            pltpu.VMEM((NCONN, O_PER_W), jnp.float32),
            pltpu.VMEM((WCHUNK,), jnp.float32),
        ],
        compiler_params=pltpu.CompilerParams(needs_layout_passes=False),
    )
    def k(conns_hbm, w_hbm, wt_hbm, conns_v, w_v, wflat):
        wid = lax.axis_index("s") * 2 + lax.axis_index("c")
        pltpu.sync_copy(conns_hbm.at[wid], conns_v)
        pltpu.sync_copy(w_hbm.at[wid], w_v)

        zero = jnp.zeros((16,), jnp.float32)

        def zbody(i, carry):
            wflat[pl.ds(i * 16, 16)] = zero
            return carry

        lax.fori_loop(0, WCHUNK // 16, zbody, 0)

        lane = lax.iota(jnp.int32, 16)
        for g in range(O_PER_W // 16):
            base_idx = (g * 16 + lane) * INPUTSIZE
            for kk in range(KTOT):
                cv = conns_v[kk, pl.ds(g * 16, 16)]
                wv = w_v[kk % NCONN, pl.ds(g * 16, 16)]
                plsc.addupdate_scatter(wflat, [base_idx + cv], wv)

        pltpu.sync_copy(wflat, wt_hbm.at[pl.ds(wid * WCHUNK, WCHUNK)])

    return k(conns_prep, weights_prep)


_BM = 512
_BN = 512


def _mm_body(x_ref, wt_ref, b_ref, xr_ref, o_ref):
    acc = lax.dot_general(
        x_ref[...], wt_ref[...],
        (((1,), (1,)), ((), ())),
        preferred_element_type=jnp.float32,
    )
    o_ref[...] = acc + b_ref[...] + xr_ref[...]


def _mm(x, wt, bias2d):
    return pl.pallas_call(
        _mm_body,
        grid=(BATCH // _BM, OUTPUTSIZE // _BN),
        in_specs=[
            pl.BlockSpec((_BM, INPUTSIZE), lambda i, j: (i, 0)),
            pl.BlockSpec((_BN, INPUTSIZE), lambda i, j: (j, 0)),
            pl.BlockSpec((1, _BN), lambda i, j: (0, j)),
            pl.BlockSpec((_BM, _BN), lambda i, j: (i, j)),
        ],
        out_specs=pl.BlockSpec((_BM, _BN), lambda i, j: (i, j)),
        out_shape=jax.ShapeDtypeStruct((BATCH, OUTPUTSIZE), jnp.float32),
    )(x, wt, bias2d, x)


def kernel(x, weights, bias, conns):
    conns_prep = conns.reshape(NW, O_PER_W, KTOT).transpose(0, 2, 1)
    weights_prep = weights.reshape(NW, O_PER_W, NCONN).transpose(0, 2, 1)
    wt = _sc_build_wt(conns_prep, weights_prep).reshape(OUTPUTSIZE, INPUTSIZE)
    return _mm(x, wt, bias.reshape(1, OUTPUTSIZE))


# speedup vs baseline: 6.5838x; 1.3800x over previous
import functools

import jax
import jax.numpy as jnp
from jax import lax
from jax.experimental import pallas as pl
from jax.experimental.pallas import tpu as pltpu
from jax.experimental.pallas import tpu_sc as plsc

INPUTSIZE = 2048
OUTPUTSIZE = 1024
BATCH = 2048
NCONN = 16
KTOT = 32

NW = 32
O_PER_W = OUTPUTSIZE // NW
WCHUNK = O_PER_W * INPUTSIZE


def _sc_build_wt(conns_prep, weights_prep):
    mesh = plsc.VectorSubcoreMesh(core_axis_name="c", subcore_axis_name="s")

    @functools.partial(
        pl.kernel,
        mesh=mesh,
        out_type=jax.ShapeDtypeStruct((OUTPUTSIZE * INPUTSIZE,), jnp.float32),
        scratch_types=[
            pltpu.VMEM((KTOT, O_PER_W), jnp.int32),
            pltpu.VMEM((NCONN, O_PER_W), jnp.float32),
            pltpu.VMEM((WCHUNK,), jnp.float32),
            pltpu.SemaphoreType.DMA,
        ],
        compiler_params=pltpu.CompilerParams(needs_layout_passes=False),
    )
    def k(conns_hbm, w_hbm, wt_hbm, conns_v, w_v, wflat, sem):
        wid = lax.axis_index("s") * 2 + lax.axis_index("c")
        cp_c = pltpu.async_copy(conns_hbm.at[wid], conns_v, sem)
        cp_w = pltpu.async_copy(w_hbm.at[wid], w_v, sem)

        zero = jnp.zeros((16,), jnp.float32)

        @plsc.parallel_loop(0, WCHUNK, step=16, unroll=8)
        def _zero(i):
            wflat[pl.ds(i, 16)] = zero

        cp_c.wait()
        cp_w.wait()

        lane = lax.iota(jnp.int32, 16)
        for g in range(O_PER_W // 16):
            base_idx = (g * 16 + lane) * INPUTSIZE
            for kk in range(KTOT):
                cv = conns_v[kk, pl.ds(g * 16, 16)]
                wv = w_v[kk % NCONN, pl.ds(g * 16, 16)]
                plsc.addupdate_scatter(wflat, [base_idx + cv], wv)

        pltpu.sync_copy(wflat, wt_hbm.at[pl.ds(wid * WCHUNK, WCHUNK)])

    return k(conns_prep, weights_prep)


_BM = 1024


def _mm_body(xb_ref, wt_ref, b_ref, xr_ref, o_ref):
    wb = wt_ref[...].astype(jnp.bfloat16)
    acc = lax.dot_general(
        xb_ref[...], wb,
        (((1,), (1,)), ((), ())),
        preferred_element_type=jnp.float32,
    )
    o_ref[...] = acc + b_ref[...] + xr_ref[...]


def _mm(xb, wt, bias2d, x):
    return pl.pallas_call(
        _mm_body,
        grid=(BATCH // _BM,),
        in_specs=[
            pl.BlockSpec((_BM, INPUTSIZE), lambda i: (i, 0)),
            pl.BlockSpec((OUTPUTSIZE, INPUTSIZE), lambda i: (0, 0)),
            pl.BlockSpec((1, OUTPUTSIZE), lambda i: (0, 0)),
            pl.BlockSpec((_BM, OUTPUTSIZE), lambda i: (i, 0)),
        ],
        out_specs=pl.BlockSpec((_BM, OUTPUTSIZE), lambda i: (i, 0)),
        out_shape=jax.ShapeDtypeStruct((BATCH, OUTPUTSIZE), jnp.float32),
    )(xb, wt, bias2d, x)


def kernel(x, weights, bias, conns):
    conns_prep = conns.reshape(NW, O_PER_W, KTOT).transpose(0, 2, 1)
    weights_prep = weights.reshape(NW, O_PER_W, NCONN).transpose(0, 2, 1)
    xb = x.astype(jnp.bfloat16)
    wt = _sc_build_wt(conns_prep, weights_prep).reshape(OUTPUTSIZE, INPUTSIZE)
    return _mm(xb, wt, bias.reshape(1, OUTPUTSIZE), x)


# speedup vs baseline: 7.4486x; 1.1314x over previous
import functools

import jax
import jax.numpy as jnp
from jax import lax
from jax.experimental import pallas as pl
from jax.experimental.pallas import tpu as pltpu
from jax.experimental.pallas import tpu_sc as plsc

INPUTSIZE = 2048
OUTPUTSIZE = 1024
BATCH = 2048
NCONN = 16
KTOT = 32

NW = 32
O_PER_W = OUTPUTSIZE // NW
WCHUNK = O_PER_W * INPUTSIZE


def _sc_build_wt(conns_prep, weights_prep):
    mesh = plsc.VectorSubcoreMesh(core_axis_name="c", subcore_axis_name="s")

    @functools.partial(
        pl.kernel,
        mesh=mesh,
        out_type=jax.ShapeDtypeStruct((NW, O_PER_W, INPUTSIZE), jnp.float32),
        scratch_types=[
            pltpu.VMEM((KTOT, O_PER_W), jnp.int32),
            pltpu.VMEM((NCONN, O_PER_W), jnp.float32),
            pltpu.VMEM((O_PER_W, INPUTSIZE), jnp.float32),
            pltpu.SemaphoreType.DMA,
        ],
        compiler_params=pltpu.CompilerParams(needs_layout_passes=False),
    )
    def k(conns_hbm, w_hbm, wt_hbm, conns_v, w_v, wchunk, sem):
        wid = lax.axis_index("s") * 2 + lax.axis_index("c")
        cp_c = pltpu.async_copy(conns_hbm.at[wid], conns_v, sem)
        cp_w = pltpu.async_copy(w_hbm.at[wid], w_v, sem)

        zero = jnp.zeros((16,), jnp.float32)

        @plsc.parallel_loop(0, WCHUNK, step=16, unroll=8)
        def _zero(i):
            r = lax.shift_right_logical(i, 11)
            c = i - lax.shift_left(r, 11)
            wchunk[r, pl.ds(c, 16)] = zero

        cp_c.wait()
        cp_w.wait()

        lane = lax.iota(jnp.int32, 16)
        one = jnp.ones((16,), jnp.float32)
        for g in range(O_PER_W // 16):
            olocal = g * 16 + lane
            plsc.addupdate_scatter(wchunk, [olocal, wid * O_PER_W + olocal], one)
            for kk in range(KTOT):
                cv = conns_v[kk, pl.ds(g * 16, 16)]
                wv = w_v[kk % NCONN, pl.ds(g * 16, 16)]
                plsc.addupdate_scatter(wchunk, [olocal, cv], wv)

        pltpu.sync_copy(wchunk, wt_hbm.at[wid])

    return k(conns_prep, weights_prep)


_BM = 512


def _mm_body(xb_ref, wt_ref, b_ref, o_ref):
    acc = lax.dot_general(
        xb_ref[...], wt_ref[...],
        (((1,), (1,)), ((), ())),
        preferred_element_type=jnp.float32,
    )
    o_ref[...] = acc + b_ref[...]


def _mm(xb, wtb, bias2d):
    return pl.pallas_call(
        _mm_body,
        grid=(BATCH // _BM,),
        in_specs=[
            pl.BlockSpec((_BM, INPUTSIZE), lambda i: (i, 0)),
            pl.BlockSpec((OUTPUTSIZE, INPUTSIZE), lambda i: (0, 0)),
            pl.BlockSpec((1, OUTPUTSIZE), lambda i: (0, 0)),
        ],
        out_specs=pl.BlockSpec((_BM, OUTPUTSIZE), lambda i: (i, 0)),
        out_shape=jax.ShapeDtypeStruct((BATCH, OUTPUTSIZE), jnp.float32),
    )(xb, wtb, bias2d)


def kernel(x, weights, bias, conns):
    conns_prep = conns.reshape(NW, O_PER_W, KTOT).transpose(0, 2, 1)
    weights_prep = weights.reshape(NW, O_PER_W, NCONN).transpose(0, 2, 1)
    xb = x.astype(jnp.bfloat16)
    wt = _sc_build_wt(conns_prep, weights_prep)
    wtb = wt.reshape(OUTPUTSIZE, INPUTSIZE).astype(jnp.bfloat16)
    return _mm(xb, wtb, bias.reshape(1, OUTPUTSIZE))


# speedup vs baseline: 7.9783x; 1.0711x over previous
import functools

import jax
import jax.numpy as jnp
from jax import lax
from jax.experimental import pallas as pl
from jax.experimental.pallas import tpu as pltpu
from jax.experimental.pallas import tpu_sc as plsc

INPUTSIZE = 2048
OUTPUTSIZE = 1024
BATCH = 2048
NCONN = 16
KTOT = 32

NW = 32
O_PER_W = OUTPUTSIZE // NW
WCHUNK = O_PER_W * INPUTSIZE


def _sc_build_wt(conns, weights):
    mesh = plsc.VectorSubcoreMesh(core_axis_name="c", subcore_axis_name="s")

    @functools.partial(
        pl.kernel,
        mesh=mesh,
        out_type=jax.ShapeDtypeStruct((NW, O_PER_W, INPUTSIZE), jnp.float32),
        scratch_types=[
            pltpu.VMEM((O_PER_W, KTOT), jnp.int32),
            pltpu.VMEM((O_PER_W, NCONN), jnp.float32),
            pltpu.VMEM((O_PER_W, INPUTSIZE), jnp.float32),
            pltpu.SemaphoreType.DMA,
        ],
        compiler_params=pltpu.CompilerParams(needs_layout_passes=False),
    )
    def k(conns_hbm, w_hbm, wt_hbm, conns_v, w_v, wchunk, sem):
        wid = lax.axis_index("s") * 2 + lax.axis_index("c")
        obase = wid * O_PER_W
        cp_c = pltpu.async_copy(conns_hbm.at[pl.ds(obase, O_PER_W), :], conns_v, sem)
        cp_w = pltpu.async_copy(w_hbm.at[pl.ds(obase, O_PER_W), :], w_v, sem)

        zero = jnp.zeros((16,), jnp.float32)

        @plsc.parallel_loop(0, WCHUNK, step=16, unroll=8)
        def _zero(i):
            r = lax.shift_right_logical(i, 11)
            c = i - lax.shift_left(r, 11)
            wchunk[r, pl.ds(c, 16)] = zero

        cp_c.wait()
        cp_w.wait()

        lane = lax.iota(jnp.int32, 16)
        one = jnp.ones((16,), jnp.float32)
        for g in range(O_PER_W // 16):
            olocal = g * 16 + lane
            plsc.addupdate_scatter(wchunk, [olocal, obase + olocal], one)
            for kk in range(KTOT):
                kvec = jnp.full((16,), kk, jnp.int32)
                wvec = jnp.full((16,), kk % NCONN, jnp.int32)
                cv = plsc.load_gather(conns_v, [olocal, kvec])
                wv = plsc.load_gather(w_v, [olocal, wvec])
                plsc.addupdate_scatter(wchunk, [olocal, cv], wv)

        pltpu.sync_copy(wchunk, wt_hbm.at[wid])

    return k(conns, weights)


_BM = 256


def _mm_body(xb_ref, wt_ref, b_ref, o_ref, wtb_scr):
    @pl.when(pl.program_id(0) == 0)
    def _():
        wtb_scr[...] = wt_ref[...].astype(jnp.bfloat16)

    acc = lax.dot_general(
        xb_ref[...], wtb_scr[...],
        (((1,), (1,)), ((), ())),
        preferred_element_type=jnp.float32,
    )
    o_ref[...] = acc + b_ref[...]


def _mm(xb, wt, bias2d):
    return pl.pallas_call(
        _mm_body,
        grid=(BATCH // _BM,),
        in_specs=[
            pl.BlockSpec((_BM, INPUTSIZE), lambda i: (i, 0)),
            pl.BlockSpec((OUTPUTSIZE, INPUTSIZE), lambda i: (0, 0)),
            pl.BlockSpec((1, OUTPUTSIZE), lambda i: (0, 0)),
        ],
        out_specs=pl.BlockSpec((_BM, OUTPUTSIZE), lambda i: (i, 0)),
        out_shape=jax.ShapeDtypeStruct((BATCH, OUTPUTSIZE), jnp.float32),
        scratch_shapes=[pltpu.VMEM((OUTPUTSIZE, INPUTSIZE), jnp.bfloat16)],
    )(xb, wt, bias2d)


def kernel(x, weights, bias, conns):
    xb = x.astype(jnp.bfloat16)
    wt = _sc_build_wt(conns, weights).reshape(OUTPUTSIZE, INPUTSIZE)
    return _mm(xb, wt, bias.reshape(1, OUTPUTSIZE))


# speedup vs baseline: 8.0569x; 1.0098x over previous
import functools

import jax
import jax.numpy as jnp
from jax import lax
from jax.experimental import pallas as pl
from jax.experimental.pallas import tpu as pltpu
from jax.experimental.pallas import tpu_sc as plsc

INPUTSIZE = 2048
OUTPUTSIZE = 1024
BATCH = 2048
NCONN = 16
KTOT = 32

NW = 32
O_PER_W = OUTPUTSIZE // NW
WCHUNK = O_PER_W * INPUTSIZE


def _sc_build_wt(conns, weights):
    mesh = plsc.VectorSubcoreMesh(core_axis_name="c", subcore_axis_name="s")

    @functools.partial(
        pl.kernel,
        mesh=mesh,
        out_type=jax.ShapeDtypeStruct((NW, O_PER_W, INPUTSIZE), jnp.float32),
        scratch_types=[
            pltpu.VMEM((O_PER_W, KTOT), jnp.int32),
            pltpu.VMEM((O_PER_W, NCONN), jnp.float32),
            pltpu.VMEM((O_PER_W, INPUTSIZE), jnp.float32),
            pltpu.SemaphoreType.DMA,
        ],
        compiler_params=pltpu.CompilerParams(needs_layout_passes=False),
    )
    def k(conns_hbm, w_hbm, wt_hbm, conns_v, w_v, wchunk, sem):
        wid = lax.axis_index("s") * 2 + lax.axis_index("c")
        obase = wid * O_PER_W
        cp_c = pltpu.async_copy(conns_hbm.at[pl.ds(obase, O_PER_W), :], conns_v, sem)
        cp_w = pltpu.async_copy(w_hbm.at[pl.ds(obase, O_PER_W), :], w_v, sem)

        zero = jnp.zeros((16,), jnp.float32)

        @plsc.parallel_loop(0, WCHUNK, step=16, unroll=8)
        def _zero(i):
            r = lax.shift_right_logical(i, 11)
            c = i - lax.shift_left(r, 11)
            wchunk[r, pl.ds(c, 16)] = zero

        cp_c.wait()
        cp_w.wait()

        lane = lax.iota(jnp.int32, 16)
        one = jnp.ones((16,), jnp.float32)
        for g in range(O_PER_W // 16):
            olocal = g * 16 + lane
            plsc.addupdate_scatter(wchunk, [olocal, obase + olocal], one)
            for kk in range(KTOT):
                kvec = jnp.full((16,), kk, jnp.int32)
                wvec = jnp.full((16,), kk % NCONN, jnp.int32)
                cv = plsc.load_gather(conns_v, [olocal, kvec])
                wv = plsc.load_gather(w_v, [olocal, wvec])
                plsc.addupdate_scatter(wchunk, [olocal, cv], wv)

        pltpu.sync_copy(wchunk, wt_hbm.at[wid])

    return k(conns, weights)


_BM = 256


def _mm_body(xb_ref, wt_ref, b_ref, o_ref):
    acc = lax.dot_general(
        xb_ref[...], wt_ref[...],
        (((1,), (1,)), ((), ())),
        preferred_element_type=jnp.float32,
    )
    o_ref[...] = acc + b_ref[...]


def _mm(xb, wt, bias2d):
    return pl.pallas_call(
        _mm_body,
        grid=(BATCH // _BM,),
        in_specs=[
            pl.BlockSpec((_BM, INPUTSIZE), lambda i: (i, 0)),
            pl.BlockSpec((OUTPUTSIZE, INPUTSIZE), lambda i: (0, 0)),
            pl.BlockSpec((1, OUTPUTSIZE), lambda i: (0, 0)),
        ],
        out_specs=pl.BlockSpec((_BM, OUTPUTSIZE), lambda i: (i, 0)),
        out_shape=jax.ShapeDtypeStruct((BATCH, OUTPUTSIZE), jnp.float32),
    )(xb, wt, bias2d)


def kernel(x, weights, bias, conns):
    xb = x.astype(jnp.bfloat16)
    wt = _sc_build_wt(conns, weights).reshape(OUTPUTSIZE, INPUTSIZE)
    return _mm(xb, wt, bias.reshape(1, OUTPUTSIZE))


# speedup vs baseline: 8.2947x; 1.0295x over previous
import functools

import jax
import jax.numpy as jnp
from jax import lax
from jax.experimental import pallas as pl
from jax.experimental.pallas import tpu as pltpu
from jax.experimental.pallas import tpu_sc as plsc

INPUTSIZE = 2048
OUTPUTSIZE = 1024
BATCH = 2048
NCONN = 16
KTOT = 32

NW = 32
O_PER_W = OUTPUTSIZE // NW
WCHUNK = O_PER_W * INPUTSIZE


def _sc_build_wt(conns, weights):
    mesh = plsc.VectorSubcoreMesh(core_axis_name="c", subcore_axis_name="s")

    @functools.partial(
        pl.kernel,
        mesh=mesh,
        out_type=jax.ShapeDtypeStruct((NW, O_PER_W, INPUTSIZE), jnp.float32),
        scratch_types=[
            pltpu.VMEM((O_PER_W, KTOT), jnp.int32),
            pltpu.VMEM((O_PER_W, NCONN), jnp.float32),
            pltpu.VMEM((O_PER_W, INPUTSIZE), jnp.float32),
            pltpu.SemaphoreType.DMA,
        ],
        compiler_params=pltpu.CompilerParams(needs_layout_passes=False),
    )
    def k(conns_hbm, w_hbm, wt_hbm, conns_v, w_v, wchunk, sem):
        wid = lax.axis_index("s") * 2 + lax.axis_index("c")
        obase = wid * O_PER_W
        cp_c = pltpu.async_copy(conns_hbm.at[pl.ds(obase, O_PER_W), :], conns_v, sem)
        cp_w = pltpu.async_copy(w_hbm.at[pl.ds(obase, O_PER_W), :], w_v, sem)

        zero = jnp.zeros((16,), jnp.float32)

        @plsc.parallel_loop(0, WCHUNK, step=16, unroll=8)
        def _zero(i):
            r = lax.shift_right_logical(i, 11)
            c = i - lax.shift_left(r, 11)
            wchunk[r, pl.ds(c, 16)] = zero

        cp_c.wait()
        cp_w.wait()

        lane = lax.iota(jnp.int32, 16)
        one = jnp.ones((16,), jnp.float32)
        for g in range(O_PER_W // 16):
            olocal = g * 16 + lane
            plsc.addupdate_scatter(wchunk, [olocal, obase + olocal], one)
            for kk in range(KTOT):
                kvec = jnp.full((16,), kk, jnp.int32)
                wvec = jnp.full((16,), kk % NCONN, jnp.int32)
                cv = plsc.load_gather(conns_v, [olocal, kvec])
                wv = plsc.load_gather(w_v, [olocal, wvec])
                plsc.addupdate_scatter(wchunk, [olocal, cv], wv)

        pltpu.sync_copy(wchunk, wt_hbm.at[wid])

    return k(conns, weights)


_BM = 512


def _mm_body(xb_ref, wt_ref, b_ref, o_ref):
    acc = lax.dot_general(
        xb_ref[...], wt_ref[...],
        (((1,), (1,)), ((), ())),
        preferred_element_type=jnp.float32,
    )
    o_ref[...] = acc + b_ref[...]


def _mm(xb, wt, bias2d):
    return pl.pallas_call(
        _mm_body,
        grid=(BATCH // _BM,),
        in_specs=[
            pl.BlockSpec((_BM, INPUTSIZE), lambda i: (i, 0)),
            pl.BlockSpec((OUTPUTSIZE, INPUTSIZE), lambda i: (0, 0)),
            pl.BlockSpec((1, OUTPUTSIZE), lambda i: (0, 0)),
        ],
        out_specs=pl.BlockSpec((_BM, OUTPUTSIZE), lambda i: (i, 0)),
        out_shape=jax.ShapeDtypeStruct((BATCH, OUTPUTSIZE), jnp.float32),
    )(xb, wt, bias2d)


def kernel(x, weights, bias, conns):
    xb = x.astype(jnp.bfloat16)
    wt = _sc_build_wt(conns, weights).reshape(OUTPUTSIZE, INPUTSIZE)
    return _mm(xb, wt, bias.reshape(1, OUTPUTSIZE))
